# baseline (speedup 1.0000x reference)
import jax
import jax.numpy as jnp
from jax import lax
from jax.experimental import pallas as pl
from jax.experimental.pallas import tpu as pltpu
from jax.experimental.pallas import tpu_sc as plsc

L = 16
NW = 32
VPP = 32 * 32 * 32
NPAIR = 16
TOTAL = NPAIR * VPP
VPT = TOTAL // NW
V = 256
NBLK = VPT // V
NCH = V // L
ROWS = V * 4
CH_DMA = 128
NDMA = ROWS // CH_DMA


def _axis_terms(coord):
    c0 = jnp.clip(coord, 0.0, 34.5).astype(jnp.int32)
    d = coord - c0.astype(jnp.float32)
    m0 = (c0 >= 2) & (c0 <= 33)
    m1 = (c0 >= 1) & (c0 <= 32)
    w0 = jnp.where(m0, 1.0 - d, 0.0)
    w1 = jnp.where(m1, d, 0.0)
    r0 = jnp.clip(c0 - 2, 0, 31)
    r1 = jnp.clip(c0 - 1, 0, 31)
    return w0, w1, r0, r1


def _body(table2, ys, xs, zs, out_hbm,
          cb0, cb1, idx0, idx1, w0, w1, rows0, rows1, outv0, outv1,
          csem0, csem1, gsem0, gsem1, osem0, osem1):
    cid = lax.axis_index("c")
    sid = lax.axis_index("s")
    wid = sid * 2 + cid
    q = wid // 2
    tile_base = wid * VPT
    qb = q * VPP
    iota = lax.iota(jnp.int32, L)
    coords = (ys, xs, zs)

    def fire_coords(blk, cb, sem):
        start = tile_base + blk * V
        for a in range(3):
            pltpu.async_copy(coords[a].at[pl.ds(start, V)], cb.at[a], sem)

    def drain_coords(cb, sem):
        for a in range(3):
            pltpu.make_async_copy(
                coords[a].at[pl.ds(tile_base, V)], cb.at[a], sem).wait()

    def phase1(cb, idxr, wr):
        def chunk(ch, carry):
            off = ch * L
            yc = cb[0, pl.ds(off, L)] + 2.0
            xc = cb[1, pl.ds(off, L)] + 2.0
            zc = cb[2, pl.ds(off, L)] + 2.0
            wy0, wy1, ry0, ry1 = _axis_terms(yc)
            wx0, wx1, rx0, rx1 = _axis_terms(xc)
            wz0, wz1, rz0, rz1 = _axis_terms(zc)
            del rz0, rz1
            z0 = jnp.clip(zc, 0.0, 34.5).astype(jnp.int32)
            cz = jnp.clip(z0 - 1, 0, 32)
            ay = ((ry0 << 10) + qb, (ry1 << 10) + qb)
            bx = (rx0 << 5, rx1 << 5)
            wy = (wy0, wy1)
            wx = (wx0, wx1)
            wz = (wz0, wz1)
            for yb in range(2):
                for xb in range(2):
                    c4 = yb * 2 + xb
                    wxy = wy[yb] * wx[xb]
                    idxr[pl.ds(c4 * V + off, L)] = ay[yb] + bx[xb] + cz
                    for zb in range(2):
                        m = c4 * 2 + zb
                        wr[pl.ds(m * V + off, L)] = wxy * wz[zb]
            return carry
        lax.fori_loop(0, NCH, chunk, 0)

    def fire(idxr, rowsr, sem):
        for jj in range(NDMA):
            pltpu.async_copy(
                table2.at[idxr.at[pl.ds(jj * CH_DMA, CH_DMA)]],
                rowsr.at[pl.ds(jj * CH_DMA, CH_DMA)], sem)

    def drain(idxr, rowsr, sem):
        for jj in range(NDMA):
            pltpu.make_async_copy(
                table2.at[idxr.at[pl.ds(jj * CH_DMA, CH_DMA)]],
                rowsr.at[pl.ds(jj * CH_DMA, CH_DMA)], sem).wait()

    def combine(wr, rowsr, outr):
        def chunk(ch, carry):
            off = ch * L
            wvecs = [wr[pl.ds(m * V + off, L)] for m in range(8)]
            for l in range(L):
                v = off + l
                lidx = jnp.full((L,), l, jnp.int32)
                acc = None
                for c4 in range(4):
                    r = c4 * V + v
                    h0 = rowsr[r, pl.ds(0, L)]
                    h1 = rowsr[r, pl.ds(L, L)]
                    ws0 = wvecs[c4 * 2].at[lidx].get(
                        mode="promise_in_bounds")
                    ws1 = wvecs[c4 * 2 + 1].at[lidx].get(
                        mode="promise_in_bounds")
                    t = h0 * ws0
                    acc = t if acc is None else acc + t
                    acc = acc + h1 * ws1
                outr[v, :] = acc
            return carry
        lax.fori_loop(0, NCH, chunk, 0)

    def fire_out(outr, blk, sem):
        pltpu.async_copy(outr, out_hbm.at[pl.ds(tile_base + blk * V, V)], sem)

    def wait_out(outr, sem):
        pltpu.make_async_copy(
            outr, out_hbm.at[pl.ds(tile_base, V)], sem).wait()

    res = ((cb0, idx0, w0, rows0, csem0, gsem0, outv0, osem0),
           (cb1, idx1, w1, rows1, csem1, gsem1, outv1, osem1))

    fire_coords(0, cb0, csem0)
    fire_coords(1, cb1, csem1)
    drain_coords(cb0, csem0)
    phase1(cb0, idx0, w0)
    fire(idx0, rows0, gsem0)

    def sb_body(sb, carry):
        for par in range(2):
            blk = sb * 2 + par
            cb, idxr, wr, rowsr, csem, gs, outr, osem = res[par]
            ncb, nidxr, nwr, nrowsr, ncsem, ngs, _, _ = res[1 - par]

            @pl.when(blk + 2 < NBLK)
            def _():
                fire_coords(blk + 2, cb, csem)

            @pl.when(blk + 1 < NBLK)
            def _():
                drain_coords(ncb, ncsem)
                phase1(ncb, nidxr, nwr)
                fire(nidxr, nrowsr, ngs)

            drain(idxr, rowsr, gs)

            @pl.when(blk >= 2)
            def _():
                wait_out(outr, osem)

            combine(wr, rowsr, outr)
            fire_out(outr, blk, osem)
        return carry

    lax.fori_loop(0, NBLK // 2, sb_body, 0)
    wait_out(outv0, osem0)
    wait_out(outv1, osem1)


@jax.jit
def _resample(table2, ys, xs, zs):
    mesh = plsc.VectorSubcoreMesh(core_axis_name="c", subcore_axis_name="s")
    run = pl.kernel(
        _body,
        out_type=jax.ShapeDtypeStruct((TOTAL, L), jnp.float32),
        mesh=mesh,
        scratch_types=[
            pltpu.VMEM((3, V), jnp.float32),
            pltpu.VMEM((3, V), jnp.float32),
            pltpu.VMEM((ROWS,), jnp.int32),
            pltpu.VMEM((ROWS,), jnp.int32),
            pltpu.VMEM((8 * V,), jnp.float32),
            pltpu.VMEM((8 * V,), jnp.float32),
            pltpu.VMEM((ROWS, 2 * L), jnp.float32),
            pltpu.VMEM((ROWS, 2 * L), jnp.float32),
            pltpu.VMEM((V, L), jnp.float32),
            pltpu.VMEM((V, L), jnp.float32),
            pltpu.SemaphoreType.DMA,
            pltpu.SemaphoreType.DMA,
            pltpu.SemaphoreType.DMA,
            pltpu.SemaphoreType.DMA,
            pltpu.SemaphoreType.DMA,
            pltpu.SemaphoreType.DMA,
        ],
        compiler_params=pltpu.CompilerParams(
            needs_layout_passes=False, use_tc_tiling_on_sc=False),
    )
    return run(table2, ys, xs, zs)


def kernel(input_fmap, theta):
    B, P, H, W, D, C = input_fmap.shape
    flat = input_fmap.reshape(B * P * H * W * D, C)
    zrow = jnp.zeros((1, C), jnp.float32)
    table2 = jnp.concatenate(
        [jnp.concatenate([zrow, flat], axis=0),
         jnp.concatenate([flat, zrow], axis=0)], axis=1)
    x = jnp.arange(W, dtype=jnp.float32)
    y = jnp.arange(H, dtype=jnp.float32)
    z = jnp.arange(D, dtype=jnp.float32)
    x_t, y_t, z_t = jnp.meshgrid(x, y, z, indexing='xy')
    ones = jnp.ones_like(x_t.reshape(-1))
    grid = jnp.stack([y_t.reshape(-1), x_t.reshape(-1), z_t.reshape(-1), ones])
    grid = jnp.broadcast_to(grid[None, None], (B, P, 4, H * W * D))
    bg = jnp.einsum('bpij,bpjn->bpin', theta.astype(jnp.float32), grid)
    ys = bg[:, :, 0].reshape(TOTAL)
    xs = bg[:, :, 1].reshape(TOTAL)
    zs = bg[:, :, 2].reshape(TOTAL)
    out = _resample(table2, ys, xs, zs)
    return out.reshape(B, P, H, W, D, C)


# speedup vs baseline: 1.0006x; 1.0006x over previous
import jax
import jax.numpy as jnp
from jax import lax
from jax.experimental import pallas as pl
from jax.experimental.pallas import tpu as pltpu
from jax.experimental.pallas import tpu_sc as plsc

L = 16
NW = 32
VPP = 32 * 32 * 32
NPAIR = 16
TOTAL = NPAIR * VPP
VPT = TOTAL // NW
V = 256
NBLK = VPT // V
NCH = V // L
ROWS = V * 4
CH_DMA = 1024
NDMA = ROWS // CH_DMA


def _axis_terms(coord):
    c0 = jnp.clip(coord, 0.0, 34.5).astype(jnp.int32)
    d = coord - c0.astype(jnp.float32)
    m0 = (c0 >= 2) & (c0 <= 33)
    m1 = (c0 >= 1) & (c0 <= 32)
    w0 = jnp.where(m0, 1.0 - d, 0.0)
    w1 = jnp.where(m1, d, 0.0)
    r0 = jnp.clip(c0 - 2, 0, 31)
    r1 = jnp.clip(c0 - 1, 0, 31)
    return w0, w1, r0, r1


def _body(table2, ys, xs, zs, out_hbm,
          cb0, cb1, idx0, idx1, w0, w1, rows0, rows1, outv0, outv1,
          csem0, csem1, gsem0, gsem1, osem0, osem1):
    cid = lax.axis_index("c")
    sid = lax.axis_index("s")
    wid = sid * 2 + cid
    q = wid // 2
    tile_base = wid * VPT
    qb = q * VPP
    iota = lax.iota(jnp.int32, L)
    coords = (ys, xs, zs)

    def fire_coords(blk, cb, sem):
        start = tile_base + blk * V
        for a in range(3):
            pltpu.async_copy(coords[a].at[pl.ds(start, V)], cb.at[a], sem)

    def drain_coords(cb, sem):
        for a in range(3):
            pltpu.make_async_copy(
                coords[a].at[pl.ds(tile_base, V)], cb.at[a], sem).wait()

    def phase1(cb, idxr, wr):
        def chunk(ch, carry):
            off = ch * L
            yc = cb[0, pl.ds(off, L)] + 2.0
            xc = cb[1, pl.ds(off, L)] + 2.0
            zc = cb[2, pl.ds(off, L)] + 2.0
            wy0, wy1, ry0, ry1 = _axis_terms(yc)
            wx0, wx1, rx0, rx1 = _axis_terms(xc)
            wz0, wz1, rz0, rz1 = _axis_terms(zc)
            del rz0, rz1
            z0 = jnp.clip(zc, 0.0, 34.5).astype(jnp.int32)
            cz = jnp.clip(z0 - 1, 0, 32)
            ay = ((ry0 << 10) + qb, (ry1 << 10) + qb)
            bx = (rx0 << 5, rx1 << 5)
            wy = (wy0, wy1)
            wx = (wx0, wx1)
            wz = (wz0, wz1)
            for yb in range(2):
                for xb in range(2):
                    c4 = yb * 2 + xb
                    wxy = wy[yb] * wx[xb]
                    idxr[pl.ds(c4 * V + off, L)] = ay[yb] + bx[xb] + cz
                    for zb in range(2):
                        m = c4 * 2 + zb
                        wr[pl.ds(m * V + off, L)] = wxy * wz[zb]
            return carry
        lax.fori_loop(0, NCH, chunk, 0)

    def fire(idxr, rowsr, sem):
        for jj in range(NDMA):
            pltpu.async_copy(
                table2.at[idxr.at[pl.ds(jj * CH_DMA, CH_DMA)]],
                rowsr.at[pl.ds(jj * CH_DMA, CH_DMA)], sem)

    def drain(idxr, rowsr, sem):
        for jj in range(NDMA):
            pltpu.make_async_copy(
                table2.at[idxr.at[pl.ds(jj * CH_DMA, CH_DMA)]],
                rowsr.at[pl.ds(jj * CH_DMA, CH_DMA)], sem).wait()

    def combine(wr, rowsr, outr):
        def chunk(ch, carry):
            off = ch * L
            wvecs = [wr[pl.ds(m * V + off, L)] for m in range(8)]
            for l in range(L):
                v = off + l
                lidx = jnp.full((L,), l, jnp.int32)
                acc = None
                for c4 in range(4):
                    r = c4 * V + v
                    h0 = rowsr[r, pl.ds(0, L)]
                    h1 = rowsr[r, pl.ds(L, L)]
                    ws0 = wvecs[c4 * 2].at[lidx].get(
                        mode="promise_in_bounds")
                    ws1 = wvecs[c4 * 2 + 1].at[lidx].get(
                        mode="promise_in_bounds")
                    t = h0 * ws0
                    acc = t if acc is None else acc + t
                    acc = acc + h1 * ws1
                outr[v, :] = acc
            return carry
        lax.fori_loop(0, NCH, chunk, 0)

    def fire_out(outr, blk, sem):
        pltpu.async_copy(outr, out_hbm.at[pl.ds(tile_base + blk * V, V)], sem)

    def wait_out(outr, sem):
        pltpu.make_async_copy(
            outr, out_hbm.at[pl.ds(tile_base, V)], sem).wait()

    res = ((cb0, idx0, w0, rows0, csem0, gsem0, outv0, osem0),
           (cb1, idx1, w1, rows1, csem1, gsem1, outv1, osem1))

    fire_coords(0, cb0, csem0)
    fire_coords(1, cb1, csem1)
    drain_coords(cb0, csem0)
    phase1(cb0, idx0, w0)
    fire(idx0, rows0, gsem0)

    def sb_body(sb, carry):
        for par in range(2):
            blk = sb * 2 + par
            cb, idxr, wr, rowsr, csem, gs, outr, osem = res[par]
            ncb, nidxr, nwr, nrowsr, ncsem, ngs, _, _ = res[1 - par]

            @pl.when(blk + 2 < NBLK)
            def _():
                fire_coords(blk + 2, cb, csem)

            @pl.when(blk + 1 < NBLK)
            def _():
                drain_coords(ncb, ncsem)
                phase1(ncb, nidxr, nwr)
                fire(nidxr, nrowsr, ngs)

            drain(idxr, rowsr, gs)

            @pl.when(blk >= 2)
            def _():
                wait_out(outr, osem)

            combine(wr, rowsr, outr)
            fire_out(outr, blk, osem)
        return carry

    lax.fori_loop(0, NBLK // 2, sb_body, 0)
    wait_out(outv0, osem0)
    wait_out(outv1, osem1)


@jax.jit
def _resample(table2, ys, xs, zs):
    mesh = plsc.VectorSubcoreMesh(core_axis_name="c", subcore_axis_name="s")
    run = pl.kernel(
        _body,
        out_type=jax.ShapeDtypeStruct((TOTAL, L), jnp.float32),
        mesh=mesh,
        scratch_types=[
            pltpu.VMEM((3, V), jnp.float32),
            pltpu.VMEM((3, V), jnp.float32),
            pltpu.VMEM((ROWS,), jnp.int32),
            pltpu.VMEM((ROWS,), jnp.int32),
            pltpu.VMEM((8 * V,), jnp.float32),
            pltpu.VMEM((8 * V,), jnp.float32),
            pltpu.VMEM((ROWS, 2 * L), jnp.float32),
            pltpu.VMEM((ROWS, 2 * L), jnp.float32),
            pltpu.VMEM((V, L), jnp.float32),
            pltpu.VMEM((V, L), jnp.float32),
            pltpu.SemaphoreType.DMA,
            pltpu.SemaphoreType.DMA,
            pltpu.SemaphoreType.DMA,
            pltpu.SemaphoreType.DMA,
            pltpu.SemaphoreType.DMA,
            pltpu.SemaphoreType.DMA,
        ],
        compiler_params=pltpu.CompilerParams(
            needs_layout_passes=False, use_tc_tiling_on_sc=False),
    )
    return run(table2, ys, xs, zs)


def kernel(input_fmap, theta):
    B, P, H, W, D, C = input_fmap.shape
    flat = input_fmap.reshape(B * P * H * W * D, C)
    zrow = jnp.zeros((1, C), jnp.float32)
    table2 = jnp.concatenate(
        [jnp.concatenate([zrow, flat], axis=0),
         jnp.concatenate([flat, zrow], axis=0)], axis=1)
    x = jnp.arange(W, dtype=jnp.float32)
    y = jnp.arange(H, dtype=jnp.float32)
    z = jnp.arange(D, dtype=jnp.float32)
    x_t, y_t, z_t = jnp.meshgrid(x, y, z, indexing='xy')
    ones = jnp.ones_like(x_t.reshape(-1))
    grid = jnp.stack([y_t.reshape(-1), x_t.reshape(-1), z_t.reshape(-1), ones])
    grid = jnp.broadcast_to(grid[None, None], (B, P, 4, H * W * D))
    bg = jnp.einsum('bpij,bpjn->bpin', theta.astype(jnp.float32), grid)
    ys = bg[:, :, 0].reshape(TOTAL)
    xs = bg[:, :, 1].reshape(TOTAL)
    zs = bg[:, :, 2].reshape(TOTAL)
    out = _resample(table2, ys, xs, zs)
    return out.reshape(B, P, H, W, D, C)


# speedup vs baseline: 2.1946x; 2.1933x over previous
import jax
import jax.numpy as jnp
from jax import lax
from jax.experimental import pallas as pl
from jax.experimental.pallas import tpu as pltpu
from jax.experimental.pallas import tpu_sc as plsc

L = 16
NW = 32
VPP = 32 * 32 * 32
NPAIR = 16
TOTAL = NPAIR * VPP
VPT = TOTAL // NW
V = 256
NBLK = VPT // V
NCH = V // L
RPC = 4 * L
ROWS = V * 4
CH_DMA = 128
NDMA = ROWS // CH_DMA


def _axis_terms(coord):
    c0 = jnp.clip(coord, 0.0, 34.5).astype(jnp.int32)
    d = coord - c0.astype(jnp.float32)
    m0 = (c0 >= 2) & (c0 <= 33)
    m1 = (c0 >= 1) & (c0 <= 32)
    w0 = jnp.where(m0, 1.0 - d, 0.0)
    w1 = jnp.where(m1, d, 0.0)
    r0 = jnp.clip(c0 - 2, 0, 31)
    r1 = jnp.clip(c0 - 1, 0, 31)
    return w0, w1, r0, r1, m0 | m1, c0


def _body(table2, ys, xs, zs, out_hbm,
          cb0, cb1, idx0, idx1, w0, w1, pos0, pos1, rows0, rows1,
          outv0, outv1,
          csem0, csem1, gsem0, gsem1, osem0, osem1):
    cid = lax.axis_index("c")
    sid = lax.axis_index("s")
    wid = sid * 2 + cid
    q = wid // 2
    tile_base = wid * VPT
    qb = q * VPP
    iota = lax.iota(jnp.int32, L)
    zero_v = jnp.zeros((L,), jnp.float32)
    coords = (ys, xs, zs)

    def fire_coords(blk, cb, sem):
        start = tile_base + blk * V
        for a in range(3):
            pltpu.async_copy(coords[a].at[pl.ds(start, V)], cb.at[a], sem)

    def drain_coords(cb, sem):
        for a in range(3):
            pltpu.make_async_copy(
                coords[a].at[pl.ds(tile_base, V)], cb.at[a], sem).wait()

    def phase1(cb, idxr, wr, posr):
        def chunk(ch, na):
            off = ch * L
            yc = cb[0, pl.ds(off, L)] + 2.0
            xc = cb[1, pl.ds(off, L)] + 2.0
            zc = cb[2, pl.ds(off, L)] + 2.0
            wy0, wy1, ry0, ry1, vy, _ = _axis_terms(yc)
            wx0, wx1, rx0, rx1, vx, _ = _axis_terms(xc)
            wz0, wz1, _, _, vz, z0 = _axis_terms(zc)
            vm = vy & vx & vz
            nvalid = jnp.sum(vm.astype(jnp.int32))

            @pl.when(nvalid > 0)
            def _():
                cz = jnp.clip(z0 - 1, 0, 32)
                ay = ((ry0 << 10) + qb, (ry1 << 10) + qb)
                bx = (rx0 << 5, rx1 << 5)
                wy = (wy0, wy1)
                wx = (wx0, wx1)
                wz = (wz0, wz1)
                base = na * RPC
                wbase = na * L
                for yb in range(2):
                    for xb in range(2):
                        c4 = yb * 2 + xb
                        wxy = wy[yb] * wx[xb]
                        idxr[pl.ds(base + c4 * L, L)] = ay[yb] + bx[xb] + cz
                        for zb in range(2):
                            m = c4 * 2 + zb
                            wr[pl.ds(m * V + wbase, L)] = wxy * wz[zb]
                posr[pl.ds(wbase, L)] = off + iota
            return na + jnp.where(nvalid > 0, 1, 0).astype(jnp.int32)
        return lax.fori_loop(0, NCH, chunk, jnp.int32(0))

    def fire(idxr, rowsr, sem, na):
        nr = na * RPC
        for jj in range(NDMA):
            @pl.when(jj * CH_DMA < nr)
            def _():
                pltpu.async_copy(
                    table2.at[idxr.at[pl.ds(jj * CH_DMA, CH_DMA)]],
                    rowsr.at[pl.ds(jj * CH_DMA, CH_DMA)], sem)

    def drain(idxr, rowsr, sem, na):
        nr = na * RPC
        for jj in range(NDMA):
            @pl.when(jj * CH_DMA < nr)
            def _():
                pltpu.make_async_copy(
                    table2.at[idxr.at[pl.ds(jj * CH_DMA, CH_DMA)]],
                    rowsr.at[pl.ds(jj * CH_DMA, CH_DMA)], sem).wait()

    def combine(wr, rowsr, outr, posr, na):
        def zchunk(ch, carry):
            off = ch * L
            for l in range(L):
                outr[off + l, :] = zero_v
            return carry
        lax.fori_loop(0, NCH, zchunk, 0)

        def chunk(s, carry):
            woff = s * L
            roff = s * RPC
            posv = posr[pl.ds(woff, L)]
            wvecs = [wr[pl.ds(m * V + woff, L)] for m in range(8)]
            for l in range(L):
                lidx = jnp.full((L,), l, jnp.int32)
                psplat = posv.at[lidx].get(mode="promise_in_bounds")
                acc = None
                for c4 in range(4):
                    r = roff + c4 * L + l
                    h0 = rowsr[r, pl.ds(0, L)]
                    h1 = rowsr[r, pl.ds(L, L)]
                    ws0 = wvecs[c4 * 2].at[lidx].get(
                        mode="promise_in_bounds")
                    ws1 = wvecs[c4 * 2 + 1].at[lidx].get(
                        mode="promise_in_bounds")
                    t = h0 * ws0
                    acc = t if acc is None else acc + t
                    acc = acc + h1 * ws1
                plsc.store_scatter(outr, [psplat, iota], acc)
            return carry
        lax.fori_loop(0, na, chunk, 0)

    def fire_out(outr, blk, sem):
        pltpu.async_copy(outr, out_hbm.at[pl.ds(tile_base + blk * V, V)], sem)

    def wait_out(outr, sem):
        pltpu.make_async_copy(
            outr, out_hbm.at[pl.ds(tile_base, V)], sem).wait()

    res = ((cb0, idx0, w0, pos0, rows0, csem0, gsem0, outv0, osem0),
           (cb1, idx1, w1, pos1, rows1, csem1, gsem1, outv1, osem1))

    def zinit(t, carry):
        idx0[pl.ds(t * L, L)] = iota * 0
        idx1[pl.ds(t * L, L)] = iota * 0
        return carry
    lax.fori_loop(0, ROWS // L, zinit, 0)

    fire_coords(0, cb0, csem0)
    fire_coords(1, cb1, csem1)
    drain_coords(cb0, csem0)
    na0 = phase1(cb0, idx0, w0, pos0)
    fire(idx0, rows0, gsem0, na0)

    def sb_body(sb, carry):
        nas = list(carry)
        for par in range(2):
            blk = sb * 2 + par
            cb, idxr, wr, posr, rowsr, csem, gs, outr, osem = res[par]
            ncb, nidxr, nwr, nposr, nrowsr, ncsem, ngs, _, _ = res[1 - par]

            @pl.when(blk + 2 < NBLK)
            def _():
                fire_coords(blk + 2, cb, csem)

            def do_next(_):
                drain_coords(ncb, ncsem)
                na = phase1(ncb, nidxr, nwr, nposr)
                fire(nidxr, nrowsr, ngs, na)
                return na

            na_next = lax.cond(blk + 1 < NBLK, do_next,
                               lambda _: jnp.int32(0), 0)

            na_cur = nas[par]
            drain(idxr, rowsr, gs, na_cur)

            @pl.when(blk >= 2)
            def _():
                wait_out(outr, osem)

            combine(wr, rowsr, outr, posr, na_cur)
            fire_out(outr, blk, osem)
            nas[1 - par] = na_next
        return tuple(nas)

    lax.fori_loop(0, NBLK // 2, sb_body, (na0, jnp.int32(0)))
    wait_out(outv0, osem0)
    wait_out(outv1, osem1)


@jax.jit
def _resample(table2, ys, xs, zs):
    mesh = plsc.VectorSubcoreMesh(core_axis_name="c", subcore_axis_name="s")
    run = pl.kernel(
        _body,
        out_type=jax.ShapeDtypeStruct((TOTAL, L), jnp.float32),
        mesh=mesh,
        scratch_types=[
            pltpu.VMEM((3, V), jnp.float32),
            pltpu.VMEM((3, V), jnp.float32),
            pltpu.VMEM((ROWS,), jnp.int32),
            pltpu.VMEM((ROWS,), jnp.int32),
            pltpu.VMEM((8 * V,), jnp.float32),
            pltpu.VMEM((8 * V,), jnp.float32),
            pltpu.VMEM((V,), jnp.int32),
            pltpu.VMEM((V,), jnp.int32),
            pltpu.VMEM((ROWS, 2 * L), jnp.float32),
            pltpu.VMEM((ROWS, 2 * L), jnp.float32),
            pltpu.VMEM((V, L), jnp.float32),
            pltpu.VMEM((V, L), jnp.float32),
            pltpu.SemaphoreType.DMA,
            pltpu.SemaphoreType.DMA,
            pltpu.SemaphoreType.DMA,
            pltpu.SemaphoreType.DMA,
            pltpu.SemaphoreType.DMA,
            pltpu.SemaphoreType.DMA,
        ],
        compiler_params=pltpu.CompilerParams(
            needs_layout_passes=False, use_tc_tiling_on_sc=False),
    )
    return run(table2, ys, xs, zs)


def kernel(input_fmap, theta):
    B, P, H, W, D, C = input_fmap.shape
    flat = input_fmap.reshape(B * P * H * W * D, C)
    zrow = jnp.zeros((1, C), jnp.float32)
    table2 = jnp.concatenate(
        [jnp.concatenate([zrow, flat], axis=0),
         jnp.concatenate([flat, zrow], axis=0)], axis=1)
    x = jnp.arange(W, dtype=jnp.float32)
    y = jnp.arange(H, dtype=jnp.float32)
    z = jnp.arange(D, dtype=jnp.float32)
    x_t, y_t, z_t = jnp.meshgrid(x, y, z, indexing='xy')
    ones = jnp.ones_like(x_t.reshape(-1))
    grid = jnp.stack([y_t.reshape(-1), x_t.reshape(-1), z_t.reshape(-1), ones])
    grid = jnp.broadcast_to(grid[None, None], (B, P, 4, H * W * D))
    bg = jnp.einsum('bpij,bpjn->bpin', theta.astype(jnp.float32), grid)
    ys = bg[:, :, 0].reshape(TOTAL)
    xs = bg[:, :, 1].reshape(TOTAL)
    zs = bg[:, :, 2].reshape(TOTAL)
    out = _resample(table2, ys, xs, zs)
    return out.reshape(B, P, H, W, D, C)


# speedup vs baseline: 2.4686x; 1.1248x over previous
import jax
import jax.numpy as jnp
from jax import lax
from jax.experimental import pallas as pl
from jax.experimental.pallas import tpu as pltpu
from jax.experimental.pallas import tpu_sc as plsc

L = 16
NW = 32
VPP = 32 * 32 * 32
NPAIR = 16
TOTAL = NPAIR * VPP
VPT = TOTAL // NW
V = 256
NBLK = VPT // V
NCH = V // L
RPC = 4 * L
ROWS = V * 4
CH_DMA = 128
NDMA = ROWS // CH_DMA


def _axis_terms(coord):
    c0 = jnp.clip(coord, 0.0, 34.5).astype(jnp.int32)
    d = coord - c0.astype(jnp.float32)
    m0 = (c0 >= 2) & (c0 <= 33)
    m1 = (c0 >= 1) & (c0 <= 32)
    w0 = jnp.where(m0, 1.0 - d, 0.0)
    w1 = jnp.where(m1, d, 0.0)
    r0 = jnp.clip(c0 - 2, 0, 31)
    r1 = jnp.clip(c0 - 1, 0, 31)
    return w0, w1, r0, r1, m0 | m1, c0


def _body(table2, bgf, out_hbm,
          cb0, cb1, idx0, idx1, w0, w1, pos0, pos1, rows0, rows1,
          outv0, outv1,
          csem0, csem1, gsem0, gsem1, osem0, osem1):
    cid = lax.axis_index("c")
    sid = lax.axis_index("s")
    wid = sid * 2 + cid
    iota = lax.iota(jnp.int32, L)
    zero_v = jnp.zeros((L,), jnp.float32)

    def gstart(blk):
        return (blk * NW + wid) * V

    def fire_coords(blk, cb, sem):
        start = gstart(blk)
        q = start >> 15
        voff = start & (VPP - 1)
        for a in range(3):
            src = pl.multiple_of(((q * 3 + a) << 15) + voff, V)
            pltpu.async_copy(bgf.at[pl.ds(src, V)], cb.at[a], sem)

    def drain_coords(cb, sem):
        for a in range(3):
            pltpu.make_async_copy(
                bgf.at[pl.ds(0, V)], cb.at[a], sem).wait()

    def phase1(cb, idxr, wr, posr, qb):
        def chunk(ch, na):
            off = ch * L
            yc = cb[0, pl.ds(off, L)] + 2.0
            xc = cb[1, pl.ds(off, L)] + 2.0
            zc = cb[2, pl.ds(off, L)] + 2.0
            wy0, wy1, ry0, ry1, vy, _ = _axis_terms(yc)
            wx0, wx1, rx0, rx1, vx, _ = _axis_terms(xc)
            wz0, wz1, _, _, vz, z0 = _axis_terms(zc)
            vm = vy & vx & vz
            nvalid = jnp.sum(vm.astype(jnp.int32))

            @pl.when(nvalid > 0)
            def _():
                cz = jnp.clip(z0 - 1, 0, 32)
                ay = ((ry0 << 10) + qb, (ry1 << 10) + qb)
                bx = (rx0 << 5, rx1 << 5)
                wy = (wy0, wy1)
                wx = (wx0, wx1)
                wz = (wz0, wz1)
                base = na * RPC
                wbase = na * L
                for yb in range(2):
                    for xb in range(2):
                        c4 = yb * 2 + xb
                        wxy = wy[yb] * wx[xb]
                        idxr[pl.ds(base + c4 * L, L)] = ay[yb] + bx[xb] + cz
                        for zb in range(2):
                            m = c4 * 2 + zb
                            wr[pl.ds(m * V + wbase, L)] = wxy * wz[zb]
                posr[pl.ds(wbase, L)] = off + iota
            return na + jnp.where(nvalid > 0, 1, 0).astype(jnp.int32)
        return lax.fori_loop(0, NCH, chunk, jnp.int32(0))

    def fire(idxr, rowsr, sem, na):
        nr = na * RPC
        for jj in range(NDMA):
            @pl.when(jj * CH_DMA < nr)
            def _():
                pltpu.async_copy(
                    table2.at[idxr.at[pl.ds(jj * CH_DMA, CH_DMA)]],
                    rowsr.at[pl.ds(jj * CH_DMA, CH_DMA)], sem)

    def drain(idxr, rowsr, sem, na):
        nr = na * RPC
        for jj in range(NDMA):
            @pl.when(jj * CH_DMA < nr)
            def _():
                pltpu.make_async_copy(
                    table2.at[idxr.at[pl.ds(jj * CH_DMA, CH_DMA)]],
                    rowsr.at[pl.ds(jj * CH_DMA, CH_DMA)], sem).wait()

    def combine(wr, rowsr, outr, posr, na):
        def zchunk(ch, carry):
            off = ch * L
            for l in range(L):
                outr[off + l, :] = zero_v
            return carry
        lax.fori_loop(0, NCH, zchunk, 0)

        def chunk(s, carry):
            woff = s * L
            roff = s * RPC
            posv = posr[pl.ds(woff, L)]
            wvecs = [wr[pl.ds(m * V + woff, L)] for m in range(8)]
            for l in range(L):
                lidx = jnp.full((L,), l, jnp.int32)
                psplat = posv.at[lidx].get(mode="promise_in_bounds")
                acc = None
                for c4 in range(4):
                    r = roff + c4 * L + l
                    h0 = rowsr[r, pl.ds(0, L)]
                    h1 = rowsr[r, pl.ds(L, L)]
                    ws0 = wvecs[c4 * 2].at[lidx].get(
                        mode="promise_in_bounds")
                    ws1 = wvecs[c4 * 2 + 1].at[lidx].get(
                        mode="promise_in_bounds")
                    t = h0 * ws0
                    acc = t if acc is None else acc + t
                    acc = acc + h1 * ws1
                plsc.store_scatter(outr, [psplat, iota], acc)
            return carry
        lax.fori_loop(0, na, chunk, 0)

    def fire_out(outr, blk, sem):
        pltpu.async_copy(outr, out_hbm.at[pl.ds(gstart(blk), V)], sem)

    def wait_out(outr, sem):
        pltpu.make_async_copy(
            outr, out_hbm.at[pl.ds(0, V)], sem).wait()

    res = ((cb0, idx0, w0, pos0, rows0, csem0, gsem0, outv0, osem0),
           (cb1, idx1, w1, pos1, rows1, csem1, gsem1, outv1, osem1))

    def zinit(t, carry):
        idx0[pl.ds(t * L, L)] = iota * 0
        idx1[pl.ds(t * L, L)] = iota * 0
        return carry
    lax.fori_loop(0, ROWS // L, zinit, 0)

    fire_coords(0, cb0, csem0)
    fire_coords(1, cb1, csem1)
    drain_coords(cb0, csem0)
    na0 = phase1(cb0, idx0, w0, pos0, (gstart(0) >> 15) << 15)
    fire(idx0, rows0, gsem0, na0)

    def sb_body(sb, carry):
        nas = list(carry)
        for par in range(2):
            blk = sb * 2 + par
            cb, idxr, wr, posr, rowsr, csem, gs, outr, osem = res[par]
            ncb, nidxr, nwr, nposr, nrowsr, ncsem, ngs, _, _ = res[1 - par]

            @pl.when(blk + 2 < NBLK)
            def _():
                fire_coords(blk + 2, cb, csem)

            def do_next(_):
                drain_coords(ncb, ncsem)
                na = phase1(ncb, nidxr, nwr, nposr,
                            (gstart(blk + 1) >> 15) << 15)
                fire(nidxr, nrowsr, ngs, na)
                return na

            na_next = lax.cond(blk + 1 < NBLK, do_next,
                               lambda _: jnp.int32(0), 0)

            na_cur = nas[par]
            drain(idxr, rowsr, gs, na_cur)

            @pl.when(blk >= 2)
            def _():
                wait_out(outr, osem)

            combine(wr, rowsr, outr, posr, na_cur)
            fire_out(outr, blk, osem)
            nas[1 - par] = na_next
        return tuple(nas)

    lax.fori_loop(0, NBLK // 2, sb_body, (na0, jnp.int32(0)))
    wait_out(outv0, osem0)
    wait_out(outv1, osem1)


@jax.jit
def _resample(table2, bgf):
    mesh = plsc.VectorSubcoreMesh(core_axis_name="c", subcore_axis_name="s")
    run = pl.kernel(
        _body,
        out_type=jax.ShapeDtypeStruct((TOTAL, L), jnp.float32),
        mesh=mesh,
        scratch_types=[
            pltpu.VMEM((3, V), jnp.float32),
            pltpu.VMEM((3, V), jnp.float32),
            pltpu.VMEM((ROWS,), jnp.int32),
            pltpu.VMEM((ROWS,), jnp.int32),
            pltpu.VMEM((8 * V,), jnp.float32),
            pltpu.VMEM((8 * V,), jnp.float32),
            pltpu.VMEM((V,), jnp.int32),
            pltpu.VMEM((V,), jnp.int32),
            pltpu.VMEM((ROWS, 2 * L), jnp.float32),
            pltpu.VMEM((ROWS, 2 * L), jnp.float32),
            pltpu.VMEM((V, L), jnp.float32),
            pltpu.VMEM((V, L), jnp.float32),
            pltpu.SemaphoreType.DMA,
            pltpu.SemaphoreType.DMA,
            pltpu.SemaphoreType.DMA,
            pltpu.SemaphoreType.DMA,
            pltpu.SemaphoreType.DMA,
            pltpu.SemaphoreType.DMA,
        ],
        compiler_params=pltpu.CompilerParams(
            needs_layout_passes=False, use_tc_tiling_on_sc=False),
    )
    return run(table2, bgf)


def kernel(input_fmap, theta):
    B, P, H, W, D, C = input_fmap.shape
    flat = input_fmap.reshape(B * P * H * W * D, C)
    zrow = jnp.zeros((1, C), jnp.float32)
    table2 = jnp.concatenate(
        [jnp.concatenate([zrow, flat], axis=0),
         jnp.concatenate([flat, zrow], axis=0)], axis=1)
    x = jnp.arange(W, dtype=jnp.float32)
    y = jnp.arange(H, dtype=jnp.float32)
    z = jnp.arange(D, dtype=jnp.float32)
    x_t, y_t, z_t = jnp.meshgrid(x, y, z, indexing='xy')
    ones = jnp.ones_like(x_t.reshape(-1))
    grid = jnp.stack([y_t.reshape(-1), x_t.reshape(-1), z_t.reshape(-1), ones])
    grid = jnp.broadcast_to(grid[None, None], (B, P, 4, H * W * D))
    bg = jnp.einsum('bpij,bpjn->bpin', theta.astype(jnp.float32), grid)
    bgf = bg.reshape(NPAIR * 3 * VPP)
    out = _resample(table2, bgf)
    return out.reshape(B, P, H, W, D, C)
